# baseline (speedup 1.0000x reference)
import functools

import jax
import jax.numpy as jnp
from jax import lax
from jax.experimental import pallas as pl
from jax.experimental.pallas import tpu as pltpu
from jax.experimental.pallas import tpu_sc as plsc

E = 64
K = 512
N = 4096
G = 4
BIG = 3.0e38
BN = 2048


def _vq_body(f_ref, wt_ref, w_ref, j_ref, loss_ref, wpad_ref, wsq_ref):
    @pl.when(pl.program_id(0) == 0)
    def _():
        wt0 = wt_ref[...]
        wsq_ref[...] = jnp.sum(wt0 * wt0, axis=0, keepdims=True)
        wpad_ref[...] = jnp.pad(w_ref[...], ((0, 0), (0, _EP - E)))

    f = f_ref[...]
    wt = wt_ref[...]
    w = w_ref[...]
    prod = lax.dot_general(
        f, wt, (((1,), (0,)), ((), ())),
        preferred_element_type=jnp.float32,
        precision=lax.Precision.HIGHEST)
    d = wsq_ref[...] - 2.0 * prod

    idx = lax.broadcasted_iota(jnp.int32, (BN, K), 1).astype(jnp.float32)
    minv = jnp.min(d, axis=1, keepdims=True)
    c1 = jnp.min(jnp.where(d == minv, idx, K), axis=1, keepdims=True)
    d2 = jnp.where(idx == c1, BIG, d)
    minv2 = jnp.min(d2, axis=1, keepdims=True)
    c2 = jnp.min(jnp.where(d2 == minv2, idx, K), axis=1, keepdims=True)

    cl = jnp.minimum(c1, c2)
    ch = jnp.maximum(c1, c2)

    oh_l = (idx == cl).astype(jnp.float32)
    oh_h = (idx == ch).astype(jnp.float32)
    w_l = lax.dot_general(oh_l, w, (((1,), (0,)), ((), ())),
                          preferred_element_type=jnp.float32,
                          precision=lax.Precision.HIGHEST)
    w_h = lax.dot_general(oh_h, w, (((1,), (0,)), ((), ())),
                          preferred_element_type=jnp.float32,
                          precision=lax.Precision.HIGHEST)
    dl = jnp.sum((f - w_l) ** 2, axis=1, keepdims=True)
    dh = jnp.sum((f - w_h) ** 2, axis=1, keepdims=True)
    take_h = dh < dl
    j = jnp.where(take_h, ch, cl)
    dmin = jnp.minimum(dl, dh)

    tok_loss = 1.25 * dmin
    loss = jnp.sum(jnp.reshape(tok_loss, (BN // G, G)), axis=1,
                   keepdims=True) * (1.0 / G)

    j_ref[...] = jnp.broadcast_to(j.astype(jnp.int32), (BN, 8))
    loss_ref[...] = loss


_NC = 2
_NS = 16
_NW = _NC * _NS
_BPW = N // _NW

_EP = 128


@functools.cache
def _gather_rows_kernel():
    mesh = plsc.VectorSubcoreMesh(core_axis_name="c", subcore_axis_name="s")

    @functools.partial(
        pl.kernel,
        mesh=mesh,
        out_type=jax.ShapeDtypeStruct((N, _EP), jnp.float32),
        scratch_types=[
            pltpu.VMEM((_BPW,), jnp.int32),
            pltpu.VMEM((_BPW, _EP), jnp.float32),
            pltpu.SemaphoreType.DMA,
        ],
    )
    def _gather_rows(table_hbm, idx_hbm, out_hbm, idx_v, rows_v, sem):
        wid = lax.axis_index("s") * _NC + lax.axis_index("c")
        base = wid * _BPW
        pltpu.sync_copy(idx_hbm.at[pl.ds(base, _BPW)], idx_v)
        pltpu.async_copy(table_hbm.at[idx_v], rows_v, sem).wait()
        pltpu.sync_copy(rows_v, out_hbm.at[pl.ds(base, _BPW)])

    return _gather_rows


def kernel(feature, W):
    f = feature.reshape(N, E)
    wt = W.T
    j2d, loss2d, w_pad = pl.pallas_call(
        _vq_body,
        grid=(N // BN,),
        in_specs=[
            pl.BlockSpec((BN, E), lambda i: (i, 0)),
            pl.BlockSpec((E, K), lambda i: (0, 0)),
            pl.BlockSpec((K, E), lambda i: (0, 0)),
        ],
        out_specs=[
            pl.BlockSpec((BN, 8), lambda i: (i, 0)),
            pl.BlockSpec((BN // G, 1), lambda i: (i, 0)),
            pl.BlockSpec((K, _EP), lambda i: (0, 0)),
        ],
        out_shape=[
            jax.ShapeDtypeStruct((N, 8), jnp.int32),
            jax.ShapeDtypeStruct((N // G, 1), jnp.float32),
            jax.ShapeDtypeStruct((K, _EP), jnp.float32),
        ],
        scratch_shapes=[pltpu.VMEM((1, K), jnp.float32)],
    )(f, wt, W)
    j = j2d[:, 0]
    out = _gather_rows_kernel()(w_pad, j)[:, :E]
    return loss2d.reshape(N // G), out.reshape(-1, G * E)


# speedup vs baseline: 1.0267x; 1.0267x over previous
import functools

import jax
import jax.numpy as jnp
from jax import lax
from jax.experimental import pallas as pl
from jax.experimental.pallas import tpu as pltpu
from jax.experimental.pallas import tpu_sc as plsc

E = 64
K = 512
N = 4096
G = 4
BIG = 3.0e38
BN = 512


def _vq_body(f_ref, wt_ref, w_ref, j_ref, loss_ref, wpad_ref, wsq_ref):
    @pl.when(pl.program_id(0) == 0)
    def _():
        wt0 = wt_ref[...]
        wsq_ref[...] = jnp.sum(wt0 * wt0, axis=0, keepdims=True)
        wpad_ref[...] = jnp.pad(w_ref[...], ((0, 0), (0, _EP - E)))

    f = f_ref[...]
    wt = wt_ref[...]
    w = w_ref[...]
    prod = lax.dot_general(
        f, wt, (((1,), (0,)), ((), ())),
        preferred_element_type=jnp.float32,
        precision=lax.Precision.HIGHEST)
    d = wsq_ref[...] - 2.0 * prod

    idx = lax.broadcasted_iota(jnp.int32, (BN, K), 1).astype(jnp.float32)
    minv = jnp.min(d, axis=1, keepdims=True)
    c1 = jnp.min(jnp.where(d == minv, idx, K), axis=1, keepdims=True)
    d2 = jnp.where(idx == c1, BIG, d)
    minv2 = jnp.min(d2, axis=1, keepdims=True)
    c2 = jnp.min(jnp.where(d2 == minv2, idx, K), axis=1, keepdims=True)

    cl = jnp.minimum(c1, c2)
    ch = jnp.maximum(c1, c2)

    oh_l = (idx == cl).astype(jnp.float32)
    oh_h = (idx == ch).astype(jnp.float32)
    w_l = lax.dot_general(oh_l, w, (((1,), (0,)), ((), ())),
                          preferred_element_type=jnp.float32,
                          precision=lax.Precision.HIGHEST)
    w_h = lax.dot_general(oh_h, w, (((1,), (0,)), ((), ())),
                          preferred_element_type=jnp.float32,
                          precision=lax.Precision.HIGHEST)
    dl = jnp.sum((f - w_l) ** 2, axis=1, keepdims=True)
    dh = jnp.sum((f - w_h) ** 2, axis=1, keepdims=True)
    take_h = dh < dl
    j = jnp.where(take_h, ch, cl)
    dmin = jnp.minimum(dl, dh)

    tok_loss = 1.25 * dmin
    loss = jnp.sum(jnp.reshape(tok_loss, (BN // G, G)), axis=1,
                   keepdims=True) * (1.0 / G)

    j_ref[...] = jnp.broadcast_to(j.astype(jnp.int32), (BN, 8))
    loss_ref[...] = loss


_NC = 2
_NS = 16
_NW = _NC * _NS
_BPW = N // _NW

_EP = 128


@functools.cache
def _gather_rows_kernel():
    mesh = plsc.VectorSubcoreMesh(core_axis_name="c", subcore_axis_name="s")

    @functools.partial(
        pl.kernel,
        mesh=mesh,
        out_type=jax.ShapeDtypeStruct((N, _EP), jnp.float32),
        scratch_types=[
            pltpu.VMEM((_BPW,), jnp.int32),
            pltpu.VMEM((_BPW, _EP), jnp.float32),
            pltpu.SemaphoreType.DMA,
        ],
    )
    def _gather_rows(table_hbm, idx_hbm, out_hbm, idx_v, rows_v, sem):
        wid = lax.axis_index("s") * _NC + lax.axis_index("c")
        base = wid * _BPW
        pltpu.sync_copy(idx_hbm.at[pl.ds(base, _BPW)], idx_v)
        pltpu.async_copy(table_hbm.at[idx_v], rows_v, sem).wait()
        pltpu.sync_copy(rows_v, out_hbm.at[pl.ds(base, _BPW)])

    return _gather_rows


def kernel(feature, W):
    f = feature.reshape(N, E)
    wt = W.T
    j2d, loss2d, w_pad = pl.pallas_call(
        _vq_body,
        grid=(N // BN,),
        in_specs=[
            pl.BlockSpec((BN, E), lambda i: (i, 0)),
            pl.BlockSpec((E, K), lambda i: (0, 0)),
            pl.BlockSpec((K, E), lambda i: (0, 0)),
        ],
        out_specs=[
            pl.BlockSpec((BN, 8), lambda i: (i, 0)),
            pl.BlockSpec((BN // G, 1), lambda i: (i, 0)),
            pl.BlockSpec((K, _EP), lambda i: (0, 0)),
        ],
        out_shape=[
            jax.ShapeDtypeStruct((N, 8), jnp.int32),
            jax.ShapeDtypeStruct((N // G, 1), jnp.float32),
            jax.ShapeDtypeStruct((K, _EP), jnp.float32),
        ],
        scratch_shapes=[pltpu.VMEM((1, K), jnp.float32)],
    )(f, wt, W)
    j = j2d[:, 0]
    out = _gather_rows_kernel()(w_pad, j)[:, :E]
    return loss2d.reshape(N // G), out.reshape(-1, G * E)
